# Initial kernel scaffold; baseline (speedup 1.0000x reference)
import functools

import jax
import jax.numpy as jnp
from jax import lax
from jax.experimental import pallas as pl
from jax.experimental.pallas import tpu as pltpu
from jax.experimental.pallas import tpu_sc as plsc

N = 10000
E = 320000
D = 128
NC = 2
NS = 16
NW = NC * NS
EPT = E // NW
K = 128
FULL = EPT // K
TAIL = EPT - FULL * K
RPT = N // NS
ZCH = 5
ZR = RPT // ZCH

_mesh = plsc.VectorSubcoreMesh(
    core_axis_name="c", subcore_axis_name="s", num_cores=NC, num_subcores=NS
)


def _deg_body(wpad_hbm, dst_hbm, out_hbm, deg_sh, wrow_v, didx_v, wtail_v,
              dtail_v, zbuf_v):
    cid = lax.axis_index("c")
    sid = lax.axis_index("s")
    wid = cid * NS + sid
    base = wid * EPT

    @pl.loop(0, ZR)
    def _zero_buf(i):
        zbuf_v[i, :] = jnp.zeros((16,), jnp.float32)

    @pl.loop(0, ZCH)
    def _zero_shared(k):
        pltpu.sync_copy(zbuf_v, deg_sh.at[pl.ds(sid * RPT + k * ZR, ZR)])

    plsc.subcore_barrier()

    @pl.loop(0, FULL)
    def _chunk(i):
        off = base + i * K
        pltpu.sync_copy(wpad_hbm.at[pl.ds(off, K)], wrow_v)
        pltpu.sync_copy(dst_hbm.at[pl.ds(off, K)], didx_v)
        pltpu.sync_copy(wrow_v, deg_sh.at[didx_v], add=True)

    toff = base + FULL * K
    pltpu.sync_copy(wpad_hbm.at[pl.ds(toff, TAIL)], wtail_v)
    pltpu.sync_copy(dst_hbm.at[pl.ds(toff, TAIL)], dtail_v)
    pltpu.sync_copy(wtail_v, deg_sh.at[dtail_v], add=True)

    plsc.subcore_barrier()
    pltpu.sync_copy(deg_sh.at[pl.ds(sid * RPT, RPT)],
                    out_hbm.at[cid, pl.ds(sid * RPT, RPT)])


_deg_kernel = functools.partial(
    pl.kernel,
    out_type=jax.ShapeDtypeStruct((NC, N, 16), jnp.float32),
    mesh=_mesh,
    scratch_types=[
        pltpu.VMEM_SHARED((N, 16), jnp.float32),
        pltpu.VMEM((K, 16), jnp.float32),
        pltpu.VMEM((K,), jnp.int32),
        pltpu.VMEM((TAIL, 16), jnp.float32),
        pltpu.VMEM((TAIL,), jnp.int32),
        pltpu.VMEM((ZR, 16), jnp.float32),
    ],
)(_deg_body)


def _msg_body(y_hbm, src_hbm, dst_hbm, w_hbm, out_hbm, acc_sh, rows_v, sidx_v,
              didx_v, w_v, stail_v, dtail_v, wt_v):
    cid = lax.axis_index("c")
    sid = lax.axis_index("s")
    wid = cid * NS + sid
    base = wid * EPT

    @pl.loop(0, K)
    def _zero_buf(i):
        for j in range(D // 16):
            rows_v[i, pl.ds(j * 16, 16)] = jnp.zeros((16,), jnp.float32)

    @pl.loop(0, ZCH)
    def _zero_shared(k):
        pltpu.sync_copy(rows_v.at[pl.ds(0, ZR)],
                        acc_sh.at[pl.ds(sid * RPT + k * ZR, ZR)])

    plsc.subcore_barrier()

    @pl.loop(0, FULL)
    def _chunk(i):
        off = base + i * K
        pltpu.sync_copy(src_hbm.at[pl.ds(off, K)], sidx_v)
        pltpu.sync_copy(dst_hbm.at[pl.ds(off, K)], didx_v)
        pltpu.sync_copy(w_hbm.at[pl.ds(off, K)], w_v)
        pltpu.sync_copy(y_hbm.at[sidx_v], rows_v)

        @pl.loop(0, K)
        def _scale(e):
            we = w_v[e]
            for j in range(D // 16):
                sl = (e, pl.ds(j * 16, 16))
                rows_v[sl] = rows_v[sl] * we

        pltpu.sync_copy(rows_v, acc_sh.at[didx_v], add=True)

    toff = base + FULL * K
    pltpu.sync_copy(src_hbm.at[pl.ds(toff, TAIL)], stail_v)
    pltpu.sync_copy(dst_hbm.at[pl.ds(toff, TAIL)], dtail_v)
    pltpu.sync_copy(w_hbm.at[pl.ds(toff, TAIL)], wt_v)
    pltpu.sync_copy(y_hbm.at[stail_v], rows_v.at[pl.ds(0, TAIL)])

    @pl.loop(0, TAIL)
    def _scale_tail(e):
        we = wt_v[e]
        for j in range(D // 16):
            sl = (e, pl.ds(j * 16, 16))
            rows_v[sl] = rows_v[sl] * we

    pltpu.sync_copy(rows_v.at[pl.ds(0, TAIL)], acc_sh.at[dtail_v], add=True)

    plsc.subcore_barrier()

    @pl.loop(0, ZCH)
    def _copy_out(k):
        r0 = sid * RPT + k * ZR
        pltpu.sync_copy(acc_sh.at[pl.ds(r0, ZR)],
                        out_hbm.at[cid, pl.ds(r0, ZR)])


_msg_kernel = functools.partial(
    pl.kernel,
    out_type=jax.ShapeDtypeStruct((NC, N, D), jnp.float32),
    mesh=_mesh,
    scratch_types=[
        pltpu.VMEM_SHARED((N, D), jnp.float32),
        pltpu.VMEM((K, D), jnp.float32),
        pltpu.VMEM((K,), jnp.int32),
        pltpu.VMEM((K,), jnp.int32),
        pltpu.VMEM((K,), jnp.float32),
        pltpu.VMEM((TAIL,), jnp.int32),
        pltpu.VMEM((TAIL,), jnp.int32),
        pltpu.VMEM((TAIL,), jnp.float32),
    ],
)(_msg_body)


def _mm_body(x_ref, w_ref, xw_ref):
    xw_ref[...] = jnp.dot(x_ref[...], w_ref[...],
                          preferred_element_type=jnp.float32)


def _mm(x, W):
    return pl.pallas_call(
        _mm_body,
        out_shape=jax.ShapeDtypeStruct((N, D), jnp.float32),
    )(x, W)


def _scale_body(xw_ref, degp_ref, y_ref):
    s = jnp.sum(degp_ref[...], axis=0)
    deg = jnp.sum(s, axis=1, keepdims=True) + 1.0
    dis = jnp.where(deg > 0, lax.rsqrt(deg), 0.0)
    y_ref[...] = xw_ref[...] * dis


def _scale(xw, degp):
    return pl.pallas_call(
        _scale_body,
        out_shape=jax.ShapeDtypeStruct((N, D), jnp.float32),
    )(xw, degp)


def _final_body(accp_ref, y_ref, degp_ref, b_ref, o_ref):
    s = jnp.sum(degp_ref[...], axis=0)
    deg = jnp.sum(s, axis=1, keepdims=True) + 1.0
    dis = jnp.where(deg > 0, lax.rsqrt(deg), 0.0)
    agg = accp_ref[0] + accp_ref[1] + y_ref[...]
    out = dis * agg + b_ref[...]
    m = jnp.max(out, axis=0, keepdims=True)
    z = jnp.exp(out - m)
    lse = jnp.log(jnp.sum(z, axis=0, keepdims=True))
    o_ref[...] = out - m - lse


def _final(accp, y, degp, b):
    return pl.pallas_call(
        _final_body,
        out_shape=jax.ShapeDtypeStruct((N, D), jnp.float32),
    )(accp, y, degp, b)


def kernel(x, edge_index, edge_weight, W, b):
    src = edge_index[0]
    dst = edge_index[1]
    wpad = jnp.pad(edge_weight[:, None], ((0, 0), (0, 15)))
    degp = _deg_kernel(wpad, dst)
    xw = _mm(x, W)
    y = _scale(xw, degp)
    accp = _msg_kernel(y, src, dst, edge_weight)
    return _final(accp, y, degp, b)


# speedup vs baseline: 15.1160x; 15.1160x over previous
import dataclasses
import functools

import jax
import jax.numpy as jnp
from jax import lax
from jax.experimental import pallas as pl
from jax.experimental.pallas import tpu as pltpu
from jax.experimental.pallas import tpu_sc as plsc

N = 10000
E = 320000
D = 128
NC = 2
NS = 16
NW = NC * NS
EPT = E // NW
K = 128
FULL = EPT // K
TAIL = EPT - FULL * K
RB = 624
REXTRA = N - NS * RB
ZCHUNKS = ((0, 128), (128, 128), (256, 128), (384, 128), (512, 112))

_sc_params = pltpu.CompilerParams()
if "needs_layout_passes" in pltpu.CompilerParams.__dataclass_fields__:
    _sc_params = dataclasses.replace(_sc_params, needs_layout_passes=False)

_mesh = plsc.VectorSubcoreMesh(
    core_axis_name="c", subcore_axis_name="s", num_cores=NC, num_subcores=NS
)


def _deg_body(w_hbm, dst_hbm, out_hbm, deg_sh, w_v, didx_v, wtail_v,
              dtail_v, zbuf_v):
    cid = lax.axis_index("c")
    sid = lax.axis_index("s")
    wid = cid * NS + sid
    base = wid * EPT

    @pl.loop(0, RB // 16)
    def _zero_buf(i):
        zbuf_v[pl.ds(i * 16, 16)] = jnp.zeros((16,), jnp.float32)

    pltpu.sync_copy(zbuf_v, deg_sh.at[pl.ds(sid * RB, RB)])

    @pl.when(sid == NS - 1)
    def _zero_extra():
        pltpu.sync_copy(zbuf_v.at[pl.ds(0, REXTRA)],
                        deg_sh.at[pl.ds(NS * RB, REXTRA)])

    plsc.subcore_barrier()

    @pl.loop(0, FULL)
    def _chunk(i):
        off = base + i * K
        pltpu.sync_copy(w_hbm.at[pl.ds(off, K)], w_v)
        pltpu.sync_copy(dst_hbm.at[pl.ds(off, K)], didx_v)
        pltpu.sync_copy(w_v, deg_sh.at[didx_v], add=True)

    toff = base + FULL * K
    pltpu.sync_copy(w_hbm.at[pl.ds(toff, TAIL)], wtail_v)
    pltpu.sync_copy(dst_hbm.at[pl.ds(toff, TAIL)], dtail_v)
    pltpu.sync_copy(wtail_v, deg_sh.at[dtail_v], add=True)

    plsc.subcore_barrier()
    pltpu.sync_copy(deg_sh.at[pl.ds(sid * RB, RB)], zbuf_v)
    pltpu.sync_copy(zbuf_v, out_hbm.at[pl.ds(cid * N + sid * RB, RB)])

    @pl.when(sid == NS - 1)
    def _copy_extra():
        pltpu.sync_copy(deg_sh.at[pl.ds(NS * RB, REXTRA)],
                        wtail_v)
        pltpu.sync_copy(wtail_v, out_hbm.at[pl.ds(cid * N + NS * RB, REXTRA)])


_deg_kernel = functools.partial(
    pl.kernel,
    out_type=jax.ShapeDtypeStruct((NC * N,), jnp.float32),
    mesh=_mesh,
    scratch_types=[
        pltpu.VMEM_SHARED((N,), jnp.float32),
        pltpu.VMEM((K,), jnp.float32),
        pltpu.VMEM((K,), jnp.int32),
        pltpu.VMEM((TAIL,), jnp.float32),
        pltpu.VMEM((TAIL,), jnp.int32),
        pltpu.VMEM((RB,), jnp.float32),
    ],
)(_deg_body)


def _msg_body(y_hbm, src_hbm, dst_hbm, w_hbm, out_hbm, acc_sh, rows_v, sidx_v,
              didx_v, w_v, stail_v, dtail_v, wt_v):
    cid = lax.axis_index("c")
    sid = lax.axis_index("s")
    wid = cid * NS + sid
    base = wid * EPT

    @pl.loop(0, K)
    def _zero_buf(i):
        for j in range(D // 16):
            rows_v[i, pl.ds(j * 16, 16)] = jnp.zeros((16,), jnp.float32)

    for zoff, zsz in ZCHUNKS:
        pltpu.sync_copy(rows_v.at[pl.ds(0, zsz)],
                        acc_sh.at[pl.ds(sid * RB + zoff, zsz)])

    @pl.when(sid == NS - 1)
    def _zero_extra():
        pltpu.sync_copy(rows_v.at[pl.ds(0, REXTRA)],
                        acc_sh.at[pl.ds(NS * RB, REXTRA)])

    plsc.subcore_barrier()

    @pl.loop(0, FULL)
    def _chunk(i):
        off = base + i * K
        pltpu.sync_copy(src_hbm.at[pl.ds(off, K)], sidx_v)
        pltpu.sync_copy(dst_hbm.at[pl.ds(off, K)], didx_v)
        pltpu.sync_copy(w_hbm.at[pl.ds(off, K)], w_v)
        pltpu.sync_copy(y_hbm.at[sidx_v], rows_v)

        @pl.loop(0, K)
        def _scale(e):
            eidx = jnp.full((16,), e, jnp.int32)
            we = plsc.load_gather(w_v, [eidx])
            for j in range(D // 16):
                sl = (e, pl.ds(j * 16, 16))
                rows_v[sl] = rows_v[sl] * we

        pltpu.sync_copy(rows_v, acc_sh.at[didx_v], add=True)

    toff = base + FULL * K
    pltpu.sync_copy(src_hbm.at[pl.ds(toff, TAIL)], stail_v)
    pltpu.sync_copy(dst_hbm.at[pl.ds(toff, TAIL)], dtail_v)
    pltpu.sync_copy(w_hbm.at[pl.ds(toff, TAIL)], wt_v)
    pltpu.sync_copy(y_hbm.at[stail_v], rows_v.at[pl.ds(0, TAIL)])

    @pl.loop(0, TAIL)
    def _scale_tail(e):
        eidx = jnp.full((16,), e, jnp.int32)
        we = plsc.load_gather(wt_v, [eidx])
        for j in range(D // 16):
            sl = (e, pl.ds(j * 16, 16))
            rows_v[sl] = rows_v[sl] * we

    pltpu.sync_copy(rows_v.at[pl.ds(0, TAIL)], acc_sh.at[dtail_v], add=True)

    plsc.subcore_barrier()
    pltpu.sync_copy(acc_sh.at[pl.ds(sid * RB, RB)],
                    out_hbm.at[cid, pl.ds(sid * RB, RB)])

    @pl.when(sid == NS - 1)
    def _copy_extra():
        pltpu.sync_copy(acc_sh.at[pl.ds(NS * RB, REXTRA)],
                        out_hbm.at[cid, pl.ds(NS * RB, REXTRA)])


_msg_kernel = functools.partial(
    pl.kernel,
    out_type=jax.ShapeDtypeStruct((NC, N, D), jnp.float32),
    mesh=_mesh,
    scratch_types=[
        pltpu.VMEM_SHARED((N, D), jnp.float32),
        pltpu.VMEM((K, D), jnp.float32),
        pltpu.VMEM((K,), jnp.int32),
        pltpu.VMEM((K,), jnp.int32),
        pltpu.VMEM((K,), jnp.float32),
        pltpu.VMEM((TAIL,), jnp.int32),
        pltpu.VMEM((TAIL,), jnp.int32),
        pltpu.VMEM((TAIL,), jnp.float32),
    ],
    compiler_params=_sc_params,
)(_msg_body)


def _mm_body(x_ref, w_ref, xw_ref):
    xw_ref[...] = jnp.dot(x_ref[...], w_ref[...],
                          preferred_element_type=jnp.float32)


def _mm(x, W):
    return pl.pallas_call(
        _mm_body,
        out_shape=jax.ShapeDtypeStruct((N, D), jnp.float32),
    )(x, W)


def _scale_body(xw_ref, degp_ref, y_ref):
    deg = degp_ref[0] + degp_ref[1] + 1.0
    dis = jnp.where(deg > 0, lax.rsqrt(deg), 0.0)
    y_ref[...] = xw_ref[...] * dis.reshape(N, 1)


def _scale(xw, degp):
    return pl.pallas_call(
        _scale_body,
        out_shape=jax.ShapeDtypeStruct((N, D), jnp.float32),
    )(xw, degp)


def _final_body(accp_ref, y_ref, degp_ref, b_ref, o_ref):
    deg = degp_ref[0] + degp_ref[1] + 1.0
    dis = jnp.where(deg > 0, lax.rsqrt(deg), 0.0)
    agg = accp_ref[0] + accp_ref[1] + y_ref[...]
    out = dis.reshape(N, 1) * agg + b_ref[...]
    m = jnp.max(out, axis=0, keepdims=True)
    z = jnp.exp(out - m)
    lse = jnp.log(jnp.sum(z, axis=0, keepdims=True))
    o_ref[...] = out - m - lse


def _final(accp, y, degp, b):
    return pl.pallas_call(
        _final_body,
        out_shape=jax.ShapeDtypeStruct((N, D), jnp.float32),
    )(accp, y, degp, b)


def kernel(x, edge_index, edge_weight, W, b):
    src = edge_index[0]
    dst = edge_index[1]
    degp = _deg_kernel(edge_weight, dst).reshape(NC, N)
    xw = _mm(x, W)
    y = _scale(xw, degp)
    accp = _msg_kernel(y, src, dst, edge_weight)
    return _final(accp, y, degp, b)
